# Initial kernel scaffold; baseline (speedup 1.0000x reference)
import jax
import jax.numpy as jnp
from jax.experimental import pallas as pl


def kernel(x, edge_index, W0, as0, ad0, b0, W1, as1, ad1, b1, W2, as2, ad2, b2, Wp1, bp1, Wp2, bp2, qe, Wq, bq, Wk, bk, Wv, bv, Wo, bo, g1, bn1, Wf1, bf1, Wf2, bf2, g2, bn2):
    raise NotImplementedError("write your pallas kernel here")


# speedup vs baseline: 1.0403x; 1.0403x over previous
import jax
import jax.numpy as jnp
from jax.experimental import pallas as pl

_BN = 1000


def _dot(a, b):
    return jax.lax.dot_general(a, b, (((1,), (0,)), ((), ())),
                               preferred_element_type=jnp.float32)


def _dotT(a, b):
    return jax.lax.dot_general(a, b, (((1,), (1,)), ((), ())),
                               preferred_element_type=jnp.float32)


def _ln_in(x, g, b):
    m = jnp.mean(x, axis=-1, keepdims=True)
    v = jnp.mean((x - m) ** 2, axis=-1, keepdims=True)
    return (x - m) * jax.lax.rsqrt(v + 1e-5) * g + b


def _gat_proj_body(x_ref, w_ref, as_ref, ad_ref, h_ref, es_ref, ed_ref):
    h = _dot(x_ref[...], w_ref[...])
    h_ref[...] = h
    es_ref[...] = _dot(h, as_ref[...])
    ed_ref[...] = _dot(h, ad_ref[...])


def _gat_proj(x, w, a_s, a_d):
    n, din = x.shape
    hid = w.shape[1]
    nh, c = a_s.shape
    mask = ((jnp.arange(hid)[:, None] // c) == jnp.arange(128)[None, :])
    a_sm = jnp.where(mask, a_s.reshape(-1)[:, None], 0.0)
    a_dm = jnp.where(mask, a_d.reshape(-1)[:, None], 0.0)
    h, es, ed = pl.pallas_call(
        _gat_proj_body,
        grid=(n // _BN,),
        in_specs=[
            pl.BlockSpec((_BN, din), lambda i: (i, 0)),
            pl.BlockSpec((din, hid), lambda i: (0, 0)),
            pl.BlockSpec((hid, 128), lambda i: (0, 0)),
            pl.BlockSpec((hid, 128), lambda i: (0, 0)),
        ],
        out_specs=[
            pl.BlockSpec((_BN, hid), lambda i: (i, 0)),
            pl.BlockSpec((_BN, 128), lambda i: (i, 0)),
            pl.BlockSpec((_BN, 128), lambda i: (i, 0)),
        ],
        out_shape=[
            jax.ShapeDtypeStruct((n, hid), jnp.float32),
            jax.ShapeDtypeStruct((n, 128), jnp.float32),
            jax.ShapeDtypeStruct((n, 128), jnp.float32),
        ],
    )(x, w, a_sm, a_dm)
    return h, es[:, :nh], ed[:, :nh]


def _edge_phase(h, es, ed, src, dst, b):
    n, hid = h.shape
    nh = es.shape[1]
    e = es[src] + ed[dst]
    e = jnp.where(e >= 0, e, 0.2 * e)
    ex = jnp.exp(e)
    den = jax.ops.segment_sum(ex, dst, num_segments=n)
    wgt = ex / (den[dst] + 1e-16)
    msg = h.reshape(n, nh, hid // nh)[src] * wgt[:, :, None]
    out = jax.ops.segment_sum(msg, dst, num_segments=n)
    return out.reshape(n, hid) + b


def _pproj_body(h_ref, w1_ref, b1_ref, w2_ref, b2_ref, p_ref):
    t = jax.nn.gelu(_dotT(h_ref[...], w1_ref[...]) + b1_ref[...])
    p_ref[...] = _dotT(t, w2_ref[...]) + b2_ref[...]


def _pproj(h, w1, b1, w2, b2):
    n, hid = h.shape
    d = w2.shape[0]
    return pl.pallas_call(
        _pproj_body,
        grid=(n // _BN,),
        in_specs=[
            pl.BlockSpec((_BN, hid), lambda i: (i, 0)),
            pl.BlockSpec((hid, hid), lambda i: (0, 0)),
            pl.BlockSpec((1, hid), lambda i: (0, 0)),
            pl.BlockSpec((d, hid), lambda i: (0, 0)),
            pl.BlockSpec((1, d), lambda i: (0, 0)),
        ],
        out_specs=pl.BlockSpec((_BN, d), lambda i: (i, 0)),
        out_shape=jax.ShapeDtypeStruct((n, d), jnp.float32),
    )(h, w1, b1.reshape(1, -1), w2, b2.reshape(1, -1))


def _att_body(qe_ref, p_ref, wq_ref, bq_ref, wk_ref, bk_ref, wv_ref, bv_ref,
              o_ref):
    q = _dotT(qe_ref[...], wq_ref[...]) + bq_ref[0]
    k = _dotT(p_ref[...], wk_ref[...]) + bk_ref[0]
    v = _dotT(p_ref[...], wv_ref[...]) + bv_ref[0]
    dh = q.shape[-1]
    s = _dotT(q, k) * (1.0 / jnp.sqrt(jnp.float32(dh)))
    s = s - jnp.max(s, axis=-1, keepdims=True)
    se = jnp.exp(s)
    o_ref[...] = (_dot(se, v) / jnp.sum(se, axis=-1, keepdims=True))[None]


def _attention(qe, p, wq, bq, wk, bk, wv, bv, nh):
    nq, d = qe.shape
    n = p.shape[0]
    dh = d // nh
    return pl.pallas_call(
        _att_body,
        grid=(nh,),
        in_specs=[
            pl.BlockSpec((nq, d), lambda i: (0, 0)),
            pl.BlockSpec((n, d), lambda i: (0, 0)),
            pl.BlockSpec((dh, d), lambda i: (i, 0)),
            pl.BlockSpec((1, 1, dh), lambda i: (i, 0, 0)),
            pl.BlockSpec((dh, d), lambda i: (i, 0)),
            pl.BlockSpec((1, 1, dh), lambda i: (i, 0, 0)),
            pl.BlockSpec((dh, d), lambda i: (i, 0)),
            pl.BlockSpec((1, 1, dh), lambda i: (i, 0, 0)),
        ],
        out_specs=pl.BlockSpec((1, nq, dh), lambda i: (i, 0, 0)),
        out_shape=jax.ShapeDtypeStruct((nh, nq, dh), jnp.float32),
    )(qe, p, wq, bq.reshape(nh, 1, dh), wk, bk.reshape(nh, 1, dh),
      wv, bv.reshape(nh, 1, dh)).transpose(1, 0, 2).reshape(nq, d)


def _tail_body(att_ref, qe_ref, wo_ref, bo_ref, g1_ref, n1_ref,
               wf1_ref, bf1_ref, wf2_ref, bf2_ref, g2_ref, n2_ref, o_ref):
    a2 = _dotT(att_ref[...], wo_ref[...]) + bo_ref[...]
    x1 = _ln_in(qe_ref[...] + a2, g1_ref[...], n1_ref[...])
    t = jax.nn.gelu(_dotT(x1, wf1_ref[...]) + bf1_ref[...])
    f = _dotT(t, wf2_ref[...]) + bf2_ref[...]
    o_ref[...] = _ln_in(x1 + f, g2_ref[...], n2_ref[...])


def _tail(att, qe, wo, bo, g1, n1, wf1, bf1, wf2, bf2, g2, n2):
    nq, d = att.shape
    return pl.pallas_call(
        _tail_body,
        out_shape=jax.ShapeDtypeStruct((nq, d), jnp.float32),
    )(att, qe, wo, bo.reshape(1, -1), g1.reshape(1, -1), n1.reshape(1, -1),
      wf1, bf1.reshape(1, -1), wf2, bf2.reshape(1, -1),
      g2.reshape(1, -1), n2.reshape(1, -1))


def kernel(x, edge_index, W0, as0, ad0, b0, W1, as1, ad1, b1,
           W2, as2, ad2, b2, Wp1, bp1, Wp2, bp2, qe, Wq, bq, Wk, bk,
           Wv, bv, Wo, bo, g1, bn1, Wf1, bf1, Wf2, bf2, g2, bn2):
    src = edge_index[0]
    dst = edge_index[1]
    h, es, ed = _gat_proj(x, W0, as0, ad0)
    h = jax.nn.elu(_edge_phase(h, es, ed, src, dst, b0))
    h, es, ed = _gat_proj(h, W1, as1, ad1)
    h = jax.nn.elu(_edge_phase(h, es, ed, src, dst, b1))
    h, es, ed = _gat_proj(h, W2, as2, ad2)
    h = _edge_phase(h, es, ed, src, dst, b2)
    p = _pproj(h, Wp1, bp1, Wp2, bp2)
    nh = as0.shape[0]
    att = _attention(qe, p, Wq, bq, Wk, bk, Wv, bv, nh)
    out = _tail(att, qe, Wo, bo, g1, bn1, Wf1, bf1, Wf2, bf2, g2, bn2)
    return out[None]


# speedup vs baseline: 5.2548x; 5.0512x over previous
import jax
import jax.numpy as jnp
from jax.experimental import pallas as pl

_BN = 1000


def _dot(a, b):
    return jax.lax.dot_general(a, b, (((1,), (0,)), ((), ())),
                               preferred_element_type=jnp.float32)


def _dotT(a, b):
    return jax.lax.dot_general(a, b, (((1,), (1,)), ((), ())),
                               preferred_element_type=jnp.float32)


def _ln_in(x, g, b):
    m = jnp.mean(x, axis=-1, keepdims=True)
    v = jnp.mean((x - m) ** 2, axis=-1, keepdims=True)
    return (x - m) * jax.lax.rsqrt(v + 1e-5) * g + b


def _gat_proj_body(x_ref, w_ref, as_ref, ad_ref, h_ref, es_ref, ed_ref):
    h = _dot(x_ref[...], w_ref[...])
    h_ref[...] = h
    es_ref[...] = _dot(h, as_ref[...])
    ed_ref[...] = _dot(h, ad_ref[...])


def _gat_proj(x, w, a_s, a_d):
    n, din = x.shape
    hid = w.shape[1]
    nh, c = a_s.shape
    mask = ((jnp.arange(hid)[:, None] // c) == jnp.arange(128)[None, :])
    a_sm = jnp.where(mask, a_s.reshape(-1)[:, None], 0.0)
    a_dm = jnp.where(mask, a_d.reshape(-1)[:, None], 0.0)
    h, es, ed = pl.pallas_call(
        _gat_proj_body,
        grid=(n // _BN,),
        in_specs=[
            pl.BlockSpec((_BN, din), lambda i: (i, 0)),
            pl.BlockSpec((din, hid), lambda i: (0, 0)),
            pl.BlockSpec((hid, 128), lambda i: (0, 0)),
            pl.BlockSpec((hid, 128), lambda i: (0, 0)),
        ],
        out_specs=[
            pl.BlockSpec((_BN, hid), lambda i: (i, 0)),
            pl.BlockSpec((_BN, 128), lambda i: (i, 0)),
            pl.BlockSpec((_BN, 128), lambda i: (i, 0)),
        ],
        out_shape=[
            jax.ShapeDtypeStruct((n, hid), jnp.float32),
            jax.ShapeDtypeStruct((n, 128), jnp.float32),
            jax.ShapeDtypeStruct((n, 128), jnp.float32),
        ],
    )(x, w, a_sm, a_dm)
    return h, es[:, :nh], ed[:, :nh]


def _edge_phase(h, es, ed, src, dst, b):
    n, hid = h.shape
    nh = es.shape[1]
    e = es[src] + ed[dst]
    e = jnp.where(e >= 0, e, 0.2 * e)
    ex = jnp.exp(e)
    msg = (h.reshape(n, nh, hid // nh)[src] * ex[:, :, None]).reshape(-1, hid)
    acc = jax.ops.segment_sum(jnp.concatenate([msg, ex], axis=1), dst,
                              num_segments=n)
    usum = acc[:, :hid].reshape(n, nh, hid // nh)
    den = acc[:, hid:]
    out = usum / (den[:, :, None] + 1e-16)
    return out.reshape(n, hid) + b


def _pproj_body(h_ref, w1_ref, b1_ref, w2_ref, b2_ref, p_ref):
    t = jax.nn.gelu(_dotT(h_ref[...], w1_ref[...]) + b1_ref[...])
    p_ref[...] = _dotT(t, w2_ref[...]) + b2_ref[...]


def _pproj(h, w1, b1, w2, b2):
    n, hid = h.shape
    d = w2.shape[0]
    return pl.pallas_call(
        _pproj_body,
        grid=(n // _BN,),
        in_specs=[
            pl.BlockSpec((_BN, hid), lambda i: (i, 0)),
            pl.BlockSpec((hid, hid), lambda i: (0, 0)),
            pl.BlockSpec((1, hid), lambda i: (0, 0)),
            pl.BlockSpec((d, hid), lambda i: (0, 0)),
            pl.BlockSpec((1, d), lambda i: (0, 0)),
        ],
        out_specs=pl.BlockSpec((_BN, d), lambda i: (i, 0)),
        out_shape=jax.ShapeDtypeStruct((n, d), jnp.float32),
    )(h, w1, b1.reshape(1, -1), w2, b2.reshape(1, -1))


def _att_body(qe_ref, p_ref, wq_ref, bq_ref, wk_ref, bk_ref, wv_ref, bv_ref,
              o_ref):
    q = _dotT(qe_ref[...], wq_ref[...]) + bq_ref[0]
    k = _dotT(p_ref[...], wk_ref[...]) + bk_ref[0]
    v = _dotT(p_ref[...], wv_ref[...]) + bv_ref[0]
    dh = q.shape[-1]
    s = _dotT(q, k) * (1.0 / jnp.sqrt(jnp.float32(dh)))
    s = s - jnp.max(s, axis=-1, keepdims=True)
    se = jnp.exp(s)
    o_ref[...] = (_dot(se, v) / jnp.sum(se, axis=-1, keepdims=True))[None]


def _attention(qe, p, wq, bq, wk, bk, wv, bv, nh):
    nq, d = qe.shape
    n = p.shape[0]
    dh = d // nh
    return pl.pallas_call(
        _att_body,
        grid=(nh,),
        in_specs=[
            pl.BlockSpec((nq, d), lambda i: (0, 0)),
            pl.BlockSpec((n, d), lambda i: (0, 0)),
            pl.BlockSpec((dh, d), lambda i: (i, 0)),
            pl.BlockSpec((1, 1, dh), lambda i: (i, 0, 0)),
            pl.BlockSpec((dh, d), lambda i: (i, 0)),
            pl.BlockSpec((1, 1, dh), lambda i: (i, 0, 0)),
            pl.BlockSpec((dh, d), lambda i: (i, 0)),
            pl.BlockSpec((1, 1, dh), lambda i: (i, 0, 0)),
        ],
        out_specs=pl.BlockSpec((1, nq, dh), lambda i: (i, 0, 0)),
        out_shape=jax.ShapeDtypeStruct((nh, nq, dh), jnp.float32),
    )(qe, p, wq, bq.reshape(nh, 1, dh), wk, bk.reshape(nh, 1, dh),
      wv, bv.reshape(nh, 1, dh)).transpose(1, 0, 2).reshape(nq, d)


def _tail_body(att_ref, qe_ref, wo_ref, bo_ref, g1_ref, n1_ref,
               wf1_ref, bf1_ref, wf2_ref, bf2_ref, g2_ref, n2_ref, o_ref):
    a2 = _dotT(att_ref[...], wo_ref[...]) + bo_ref[...]
    x1 = _ln_in(qe_ref[...] + a2, g1_ref[...], n1_ref[...])
    t = jax.nn.gelu(_dotT(x1, wf1_ref[...]) + bf1_ref[...])
    f = _dotT(t, wf2_ref[...]) + bf2_ref[...]
    o_ref[...] = _ln_in(x1 + f, g2_ref[...], n2_ref[...])


def _tail(att, qe, wo, bo, g1, n1, wf1, bf1, wf2, bf2, g2, n2):
    nq, d = att.shape
    return pl.pallas_call(
        _tail_body,
        out_shape=jax.ShapeDtypeStruct((nq, d), jnp.float32),
    )(att, qe, wo, bo.reshape(1, -1), g1.reshape(1, -1), n1.reshape(1, -1),
      wf1, bf1.reshape(1, -1), wf2, bf2.reshape(1, -1),
      g2.reshape(1, -1), n2.reshape(1, -1))


def kernel(x, edge_index, W0, as0, ad0, b0, W1, as1, ad1, b1,
           W2, as2, ad2, b2, Wp1, bp1, Wp2, bp2, qe, Wq, bq, Wk, bk,
           Wv, bv, Wo, bo, g1, bn1, Wf1, bf1, Wf2, bf2, g2, bn2):
    src = edge_index[0]
    dst = edge_index[1]
    h, es, ed = _gat_proj(x, W0, as0, ad0)
    h = jax.nn.elu(_edge_phase(h, es, ed, src, dst, b0))
    h, es, ed = _gat_proj(h, W1, as1, ad1)
    h = jax.nn.elu(_edge_phase(h, es, ed, src, dst, b1))
    h, es, ed = _gat_proj(h, W2, as2, ad2)
    h = _edge_phase(h, es, ed, src, dst, b2)
    p = _pproj(h, Wp1, bp1, Wp2, bp2)
    nh = as0.shape[0]
    att = _attention(qe, p, Wq, bq, Wk, bk, Wv, bv, nh)
    out = _tail(att, qe, Wo, bo, g1, bn1, Wf1, bf1, Wf2, bf2, g2, bn2)
    return out[None]
